# baseline (speedup 1.0000x reference)
import functools

import jax
import jax.numpy as jnp
from jax import lax
from jax.experimental import pallas as pl
from jax.experimental.pallas import tpu as pltpu
from jax.experimental.pallas import tpu_sc as plsc

B = 32768
D = 64
V = 100000
NW = 32
ROWS_PER_W = B // NW
CHUNK = 256
N_CHUNKS = ROWS_PER_W // CHUNK

E_C = 512
N_FULL = V // E_C
TAIL = 128
V_CUT = N_FULL * E_C + TAIL


def _transpose_panel(src_hbm, out_hbm, buf1_v, buf2_v, e0, n_ent, lane64):
    pltpu.sync_copy(src_hbm.at[:, pl.ds(e0, n_ent)],
                    buf1_v.at[:, pl.ds(0, n_ent)])

    def group_body(eg, _):
        ebase = pl.multiple_of(eg * 16, 16)
        base_idx = ebase * D + lane64
        for d in range(D):
            v = buf1_v[d, pl.ds(ebase, 16)]
            plsc.store_scatter(buf2_v, [base_idx + d], v)
        return 0

    lax.fori_loop(0, n_ent // 16, group_body, 0)
    pltpu.sync_copy(buf2_v.at[pl.ds(0, n_ent * D)],
                    out_hbm.at[pl.ds(e0 * D, n_ent * D)])


def _sc_pack_body(entT_hbm, relT_hbm, ent_out_hbm, rel_out_hbm,
                  buf1_v, buf2_v):
    wid = lax.axis_index("s") * 2 + lax.axis_index("c")
    lane = lax.iota(jnp.int32, 16)
    lane64 = lane * D
    n_mine = jnp.where(wid < N_FULL % NW, N_FULL // NW + 1, N_FULL // NW)

    def chunk_body(k, _):
        c = wid + k * NW
        e0 = pl.multiple_of(c * E_C, E_C)
        _transpose_panel(entT_hbm, ent_out_hbm, buf1_v, buf2_v, e0, E_C,
                         lane64)
        _transpose_panel(relT_hbm, rel_out_hbm, buf1_v, buf2_v, e0, E_C,
                         lane64)
        return 0

    lax.fori_loop(0, n_mine, chunk_body, 0)

    @pl.when(wid == NW - 1)
    def _():
        e0 = N_FULL * E_C
        _transpose_panel(entT_hbm, ent_out_hbm, buf1_v, buf2_v, e0, TAIL,
                         lane64)
        _transpose_panel(relT_hbm, rel_out_hbm, buf1_v, buf2_v, e0, TAIL,
                         lane64)


_sc_pack = functools.partial(
    pl.kernel,
    mesh=plsc.VectorSubcoreMesh(core_axis_name="c", subcore_axis_name="s"),
    out_type=(jax.ShapeDtypeStruct((V * D,), jnp.float32),
              jax.ShapeDtypeStruct((V * D,), jnp.float32)),
    compiler_params=pltpu.CompilerParams(
        needs_layout_passes=False, use_tc_tiling_on_sc=True),
    scratch_types=[
        pltpu.VMEM((D, E_C), jnp.float32),
        pltpu.VMEM((E_C * D,), jnp.float32),
    ],
)(_sc_pack_body)


def _sc_scores_body(h_idx_hbm, t_idx_hbm, r_idx_hbm, ent_hbm, rel_hbm,
                    out_hbm, idxh_v, idxt_v, idxr_v,
                    h0_v, t0_v, r0_v, h1_v, t1_v, r1_v, tbuf_v,
                    s_v, sem0, sem1):
    wid = lax.axis_index("s") * 2 + lax.axis_index("c")
    lane = lax.iota(jnp.int32, 16)
    lane_c = lane * CHUNK
    base = wid * ROWS_PER_W

    pltpu.sync_copy(h_idx_hbm.at[pl.ds(base, ROWS_PER_W)], idxh_v)
    pltpu.sync_copy(t_idx_hbm.at[pl.ds(base, ROWS_PER_W)], idxt_v)
    pltpu.sync_copy(r_idx_hbm.at[pl.ds(base, ROWS_PER_W)], idxr_v)

    bufs = ((h0_v, t0_v, r0_v), (h1_v, t1_v, r1_v))
    sems = (sem0, sem1)

    def fire(c, slot):
        hb, tb, rb = bufs[slot]
        sl = pl.ds(c * CHUNK, CHUNK)
        return (
            pltpu.async_copy(ent_hbm.at[idxh_v.at[sl]], hb, sems[slot]),
            pltpu.async_copy(ent_hbm.at[idxt_v.at[sl]], tb, sems[slot]),
            pltpu.async_copy(rel_hbm.at[idxr_v.at[sl]], rb, sems[slot]),
        )

    def compute(c, slot):
        hb, tb, rb = bufs[slot]

        def row_body(row, _):
            q = jnp.zeros((16,), jnp.float32)
            for j in range(D // 16):
                hj = hb[row, pl.ds(j * 16, 16)]
                tj = tb[row, pl.ds(j * 16, 16)]
                rj = rb[row, pl.ds(j * 16, 16)]
                q = q + hj * rj * tj
            plsc.store_scatter(tbuf_v, [lane_c + row], q)
            return 0

        lax.fori_loop(0, CHUNK, row_body, 0, unroll=4)

        def group_body(g, _):
            gbase = pl.multiple_of(g * 16, 16)
            acc = tbuf_v[pl.ds(gbase, 16)]
            for l in range(1, 16):
                acc = acc + tbuf_v[pl.ds(gbase + l * CHUNK, 16)]
            s_v[pl.ds(c * CHUNK + gbase, 16)] = -acc
            return 0

        lax.fori_loop(0, CHUNK // 16, group_body, 0)

    pending = fire(0, 0)
    for c in range(N_CHUNKS):
        nxt = fire(c + 1, (c + 1) % 2) if c + 1 < N_CHUNKS else None
        for cp in pending:
            cp.wait()
        compute(c, c % 2)
        pending = nxt

    pltpu.sync_copy(s_v, out_hbm.at[pl.ds(base, ROWS_PER_W)])


_sc_scores = functools.partial(
    pl.kernel,
    mesh=plsc.VectorSubcoreMesh(core_axis_name="c", subcore_axis_name="s"),
    out_type=jax.ShapeDtypeStruct((B,), jnp.float32),
    compiler_params=pltpu.CompilerParams(
        needs_layout_passes=False, use_tc_tiling_on_sc=False),
    scratch_types=[
        pltpu.VMEM((ROWS_PER_W,), jnp.int32),
        pltpu.VMEM((ROWS_PER_W,), jnp.int32),
        pltpu.VMEM((ROWS_PER_W,), jnp.int32),
        pltpu.VMEM((CHUNK, D), jnp.float32),
        pltpu.VMEM((CHUNK, D), jnp.float32),
        pltpu.VMEM((CHUNK, D), jnp.float32),
        pltpu.VMEM((CHUNK, D), jnp.float32),
        pltpu.VMEM((CHUNK, D), jnp.float32),
        pltpu.VMEM((CHUNK, D), jnp.float32),
        pltpu.VMEM((16 * CHUNK,), jnp.float32),
        pltpu.VMEM((ROWS_PER_W,), jnp.float32),
        pltpu.SemaphoreType.DMA,
        pltpu.SemaphoreType.DMA,
    ],
)(_sc_scores_body)


def _loss_body(x_ref, o_ref):
    x = x_ref[...]
    p = x[:128, :]
    n = x[128:, :]

    def softplus(v):
        return jnp.maximum(v, 0.0) + jnp.log1p(jnp.exp(-jnp.abs(v)))

    tot = jnp.sum(softplus(-p)) + jnp.sum(softplus(n))
    o_ref[0] = tot / (2.0 * (B // 2))


_loss = pl.pallas_call(
    _loss_body,
    out_shape=jax.ShapeDtypeStruct((1,), jnp.float32),
    in_specs=[pl.BlockSpec(memory_space=pltpu.VMEM)],
    out_specs=pl.BlockSpec(memory_space=pltpu.SMEM),
)


def kernel(data, ent_emb, rel_emb):
    ent_flat, rel_flat = _sc_pack(ent_emb.T, rel_emb.T)
    ent_packed = lax.dynamic_update_slice(
        ent_flat.reshape(V, D), ent_emb[V_CUT:], (V_CUT, 0))
    rel_packed = lax.dynamic_update_slice(
        rel_flat.reshape(V, D), rel_emb[V_CUT:], (V_CUT, 0))
    score = _sc_scores(data[0], data[1], data[2], ent_packed, rel_packed)
    loss = _loss(score.reshape(B // 128, 128))[0]
    return loss.reshape(())


# speedup vs baseline: 1.6936x; 1.6936x over previous
import functools

import jax
import jax.numpy as jnp
from jax import lax
from jax.experimental import pallas as pl
from jax.experimental.pallas import tpu as pltpu
from jax.experimental.pallas import tpu_sc as plsc

B = 32768
D = 64
DP = 128
V = 100000
NW = 32
ROWS_PER_W = B // NW
CHUNK = 128
N_CHUNKS = ROWS_PER_W // CHUNK
TSTRIDE = CHUNK + 1

E_C = 512
N_FULL = V // E_C
TAIL = 128
V_CUT = N_FULL * E_C + TAIL


def _transpose_panel(src_hbm, out_hbm, buf1_v, buf2_v, tmp_v, e0, n_ent,
                     lane17):
    pltpu.sync_copy(src_hbm.at[:, pl.ds(e0, n_ent)],
                    buf1_v.at[:, pl.ds(0, n_ent)])

    def group_body(eg, _):
        ebase = pl.multiple_of(eg * 16, 16)
        for d0 in range(0, D, 16):
            for i in range(16):
                v = buf1_v[d0 + i, pl.ds(ebase, 16)]
                plsc.store_scatter(tmp_v, [lane17 + i], v)
            for j in range(16):
                w = tmp_v[pl.ds(j * 17, 16)]
                buf2_v[ebase + j, pl.ds(d0, 16)] = w
        return 0

    lax.fori_loop(0, n_ent // 16, group_body, 0)
    pltpu.sync_copy(buf2_v.at[pl.ds(0, n_ent), :],
                    out_hbm.at[pl.ds(e0, n_ent), :])


def _sc_pack_body(entT_hbm, relT_hbm, ent_out_hbm, rel_out_hbm,
                  buf1_v, buf2_v, tmp_v):
    wid = lax.axis_index("s") * 2 + lax.axis_index("c")
    lane = lax.iota(jnp.int32, 16)
    lane17 = lane * 17
    n_mine = jnp.where(wid < N_FULL % NW, N_FULL // NW + 1, N_FULL // NW)

    def chunk_body(k, _):
        c = wid + k * NW
        e0 = pl.multiple_of(c * E_C, E_C)
        _transpose_panel(entT_hbm, ent_out_hbm, buf1_v, buf2_v, tmp_v, e0,
                         E_C, lane17)
        _transpose_panel(relT_hbm, rel_out_hbm, buf1_v, buf2_v, tmp_v, e0,
                         E_C, lane17)
        return 0

    lax.fori_loop(0, n_mine, chunk_body, 0)

    @pl.when(wid == NW - 1)
    def _():
        e0 = N_FULL * E_C
        _transpose_panel(entT_hbm, ent_out_hbm, buf1_v, buf2_v, tmp_v, e0,
                         TAIL, lane17)
        _transpose_panel(relT_hbm, rel_out_hbm, buf1_v, buf2_v, tmp_v, e0,
                         TAIL, lane17)


_sc_pack = functools.partial(
    pl.kernel,
    mesh=plsc.VectorSubcoreMesh(core_axis_name="c", subcore_axis_name="s"),
    out_type=(jax.ShapeDtypeStruct((V, DP), jnp.float32),
              jax.ShapeDtypeStruct((V, DP), jnp.float32)),
    compiler_params=pltpu.CompilerParams(
        needs_layout_passes=False, use_tc_tiling_on_sc=True),
    scratch_types=[
        pltpu.VMEM((D, E_C), jnp.float32),
        pltpu.VMEM((E_C, DP), jnp.float32),
        pltpu.VMEM((16 * 17,), jnp.float32),
    ],
)(_sc_pack_body)


def _sc_scores_body(h_idx_hbm, t_idx_hbm, r_idx_hbm, ent_hbm, rel_hbm,
                    out_hbm, idxh_v, idxt_v, idxr_v,
                    h0_v, t0_v, r0_v, h1_v, t1_v, r1_v, tbuf_v,
                    s_v, sem0, sem1):
    wid = lax.axis_index("s") * 2 + lax.axis_index("c")
    lane = lax.iota(jnp.int32, 16)
    lane_t = lane * TSTRIDE
    base = wid * ROWS_PER_W

    pltpu.sync_copy(h_idx_hbm.at[pl.ds(base, ROWS_PER_W)], idxh_v)
    pltpu.sync_copy(t_idx_hbm.at[pl.ds(base, ROWS_PER_W)], idxt_v)
    pltpu.sync_copy(r_idx_hbm.at[pl.ds(base, ROWS_PER_W)], idxr_v)

    bufs = ((h0_v, t0_v, r0_v), (h1_v, t1_v, r1_v))
    sems = (sem0, sem1)

    def fire(c, slot):
        hb, tb, rb = bufs[slot]
        sl = pl.ds(c * CHUNK, CHUNK)
        return (
            pltpu.async_copy(ent_hbm.at[idxh_v.at[sl]], hb, sems[slot]),
            pltpu.async_copy(ent_hbm.at[idxt_v.at[sl]], tb, sems[slot]),
            pltpu.async_copy(rel_hbm.at[idxr_v.at[sl]], rb, sems[slot]),
        )

    def compute(c, slot):
        hb, tb, rb = bufs[slot]

        def row_body(row, _):
            q = jnp.zeros((16,), jnp.float32)
            for j in range(D // 16):
                hj = hb[row, pl.ds(j * 16, 16)]
                tj = tb[row, pl.ds(j * 16, 16)]
                rj = rb[row, pl.ds(j * 16, 16)]
                q = q + hj * rj * tj
            plsc.store_scatter(tbuf_v, [lane_t + row], q)
            return 0

        lax.fori_loop(0, CHUNK, row_body, 0, unroll=4)

        def group_body(g, _):
            gbase = pl.multiple_of(g * 16, 16)
            acc = tbuf_v[pl.ds(gbase, 16)]
            for l in range(1, 16):
                acc = acc + tbuf_v[pl.ds(gbase + l * TSTRIDE, 16)]
            s_v[pl.ds(c * CHUNK + gbase, 16)] = -acc
            return 0

        lax.fori_loop(0, CHUNK // 16, group_body, 0)

    pending = fire(0, 0)
    for c in range(N_CHUNKS):
        nxt = fire(c + 1, (c + 1) % 2) if c + 1 < N_CHUNKS else None
        for cp in pending:
            cp.wait()
        compute(c, c % 2)
        pending = nxt

    pltpu.sync_copy(s_v, out_hbm.at[pl.ds(base, ROWS_PER_W)])


_sc_scores = functools.partial(
    pl.kernel,
    mesh=plsc.VectorSubcoreMesh(core_axis_name="c", subcore_axis_name="s"),
    out_type=jax.ShapeDtypeStruct((B,), jnp.float32),
    compiler_params=pltpu.CompilerParams(
        needs_layout_passes=False, use_tc_tiling_on_sc=False),
    scratch_types=[
        pltpu.VMEM((ROWS_PER_W,), jnp.int32),
        pltpu.VMEM((ROWS_PER_W,), jnp.int32),
        pltpu.VMEM((ROWS_PER_W,), jnp.int32),
        pltpu.VMEM((CHUNK, DP), jnp.float32),
        pltpu.VMEM((CHUNK, DP), jnp.float32),
        pltpu.VMEM((CHUNK, DP), jnp.float32),
        pltpu.VMEM((CHUNK, DP), jnp.float32),
        pltpu.VMEM((CHUNK, DP), jnp.float32),
        pltpu.VMEM((CHUNK, DP), jnp.float32),
        pltpu.VMEM((16 * TSTRIDE,), jnp.float32),
        pltpu.VMEM((ROWS_PER_W,), jnp.float32),
        pltpu.SemaphoreType.DMA,
        pltpu.SemaphoreType.DMA,
    ],
)(_sc_scores_body)


def _loss_body(x_ref, o_ref):
    x = x_ref[...]
    p = x[:128, :]
    n = x[128:, :]

    def softplus(v):
        return jnp.maximum(v, 0.0) + jnp.log1p(jnp.exp(-jnp.abs(v)))

    tot = jnp.sum(softplus(-p)) + jnp.sum(softplus(n))
    o_ref[0] = tot / (2.0 * (B // 2))


_loss = pl.pallas_call(
    _loss_body,
    out_shape=jax.ShapeDtypeStruct((1,), jnp.float32),
    in_specs=[pl.BlockSpec(memory_space=pltpu.VMEM)],
    out_specs=pl.BlockSpec(memory_space=pltpu.SMEM),
)


def kernel(data, ent_emb, rel_emb):
    ent128, rel128 = _sc_pack(ent_emb.T, rel_emb.T)
    ent_packed = lax.dynamic_update_slice(ent128, ent_emb[V_CUT:], (V_CUT, 0))
    rel_packed = lax.dynamic_update_slice(rel128, rel_emb[V_CUT:], (V_CUT, 0))
    score = _sc_scores(data[0], data[1], data[2], ent_packed, rel_packed)
    loss = _loss(score.reshape(B // 128, 128))[0]
    return loss.reshape(())


# speedup vs baseline: 2.3058x; 1.3615x over previous
import functools

import jax
import jax.numpy as jnp
from jax import lax
from jax.experimental import pallas as pl
from jax.experimental.pallas import tpu as pltpu
from jax.experimental.pallas import tpu_sc as plsc

B = 32768
D = 64
DP = 128
V = 100000
NW = 32
ROWS_PER_W = B // NW
CHUNK = 128
N_CHUNKS = ROWS_PER_W // CHUNK
TSTRIDE = CHUNK + 1

E_C = 128
N_PANELS = V // E_C
V_CUT = N_PANELS * E_C
K_MAX = (N_PANELS + NW - 1) // NW


def _transpose_compute(buf1_v, buf2_v, tmp_v, lane17):
    def group_body(eg, _):
        ebase = pl.multiple_of(eg * 16, 16)
        for d0 in range(0, D, 16):
            for i in range(16):
                v = buf1_v[d0 + i, pl.ds(ebase, 16)]
                plsc.store_scatter(tmp_v, [lane17 + i], v)
            for j in range(16):
                w = tmp_v[pl.ds(j * 17, 16)]
                buf2_v[ebase + j, pl.ds(d0, 16)] = w
        return 0

    lax.fori_loop(0, E_C // 16, group_body, 0)


def _sc_pack_body(entT_hbm, relT_hbm, ent_out_hbm, rel_out_hbm,
                  b1e_v, b1r_v, b2e_v, b2r_v, tmp_v,
                  isem_e, isem_r, osem_e, osem_r):
    wid = lax.axis_index("s") * 2 + lax.axis_index("c")
    lane = lax.iota(jnp.int32, 16)
    lane17 = lane * 17
    n_mine = jnp.where(wid < N_PANELS % NW, K_MAX, K_MAX - 1)

    def panel_e0(k):
        return pl.multiple_of((wid + k * NW) * E_C, E_C)

    def in_desc(src_hbm, k, buf, sem):
        return pltpu.make_async_copy(
            src_hbm.at[:, pl.ds(panel_e0(k), E_C)], buf, sem)

    def out_desc(dst_hbm, k, buf, sem):
        return pltpu.make_async_copy(
            buf, dst_hbm.at[pl.ds(panel_e0(k), E_C), :], sem)

    in_desc(entT_hbm, 0, b1e_v, isem_e).start()
    in_desc(relT_hbm, 0, b1r_v, isem_r).start()

    def step(k, _):
        def do_job(src_hbm, out_hbm, b1, b2, isem, osem):
            in_desc(src_hbm, k, b1, isem).wait()

            @pl.when(k > 0)
            def _():
                out_desc(out_hbm, k - 1, b2, osem).wait()

            _transpose_compute(b1, b2, tmp_v, lane17)

            @pl.when(k + 1 < n_mine)
            def _():
                in_desc(src_hbm, k + 1, b1, isem).start()

            out_desc(out_hbm, k, b2, osem).start()

        do_job(entT_hbm, ent_out_hbm, b1e_v, b2e_v, isem_e, osem_e)
        do_job(relT_hbm, rel_out_hbm, b1r_v, b2r_v, isem_r, osem_r)
        return 0

    lax.fori_loop(0, n_mine, step, 0)
    out_desc(ent_out_hbm, n_mine - 1, b2e_v, osem_e).wait()
    out_desc(rel_out_hbm, n_mine - 1, b2r_v, osem_r).wait()


_sc_pack = functools.partial(
    pl.kernel,
    mesh=plsc.VectorSubcoreMesh(core_axis_name="c", subcore_axis_name="s"),
    out_type=(jax.ShapeDtypeStruct((V, DP), jnp.float32),
              jax.ShapeDtypeStruct((V, DP), jnp.float32)),
    compiler_params=pltpu.CompilerParams(
        needs_layout_passes=False, use_tc_tiling_on_sc=True),
    scratch_types=[
        pltpu.VMEM((D, E_C), jnp.float32),
        pltpu.VMEM((D, E_C), jnp.float32),
        pltpu.VMEM((E_C, DP), jnp.float32),
        pltpu.VMEM((E_C, DP), jnp.float32),
        pltpu.VMEM((16 * 17,), jnp.float32),
        pltpu.SemaphoreType.DMA,
        pltpu.SemaphoreType.DMA,
        pltpu.SemaphoreType.DMA,
        pltpu.SemaphoreType.DMA,
    ],
)(_sc_pack_body)


def _sc_scores_body(h_idx_hbm, t_idx_hbm, r_idx_hbm, ent_hbm, rel_hbm,
                    out_hbm, idxh_v, idxt_v, idxr_v,
                    h0_v, t0_v, r0_v, h1_v, t1_v, r1_v, tbuf_v,
                    s_v, sem0, sem1):
    wid = lax.axis_index("s") * 2 + lax.axis_index("c")
    lane = lax.iota(jnp.int32, 16)
    lane_t = lane * TSTRIDE
    base = wid * ROWS_PER_W

    pltpu.sync_copy(h_idx_hbm.at[pl.ds(base, ROWS_PER_W)], idxh_v)
    pltpu.sync_copy(t_idx_hbm.at[pl.ds(base, ROWS_PER_W)], idxt_v)
    pltpu.sync_copy(r_idx_hbm.at[pl.ds(base, ROWS_PER_W)], idxr_v)

    bufs = ((h0_v, t0_v, r0_v), (h1_v, t1_v, r1_v))
    sems = (sem0, sem1)

    def fire(c, slot):
        hb, tb, rb = bufs[slot]
        sl = pl.ds(c * CHUNK, CHUNK)
        return (
            pltpu.async_copy(ent_hbm.at[idxh_v.at[sl]], hb, sems[slot]),
            pltpu.async_copy(ent_hbm.at[idxt_v.at[sl]], tb, sems[slot]),
            pltpu.async_copy(rel_hbm.at[idxr_v.at[sl]], rb, sems[slot]),
        )

    def compute(c, slot):
        hb, tb, rb = bufs[slot]

        def row_body(row, _):
            q = jnp.zeros((16,), jnp.float32)
            for j in range(D // 16):
                hj = hb[row, pl.ds(j * 16, 16)]
                tj = tb[row, pl.ds(j * 16, 16)]
                rj = rb[row, pl.ds(j * 16, 16)]
                q = q + hj * rj * tj
            plsc.store_scatter(tbuf_v, [lane_t + row], q)
            return 0

        lax.fori_loop(0, CHUNK, row_body, 0, unroll=4)

        def group_body(g, _):
            gbase = pl.multiple_of(g * 16, 16)
            acc = tbuf_v[pl.ds(gbase, 16)]
            for l in range(1, 16):
                acc = acc + tbuf_v[pl.ds(gbase + l * TSTRIDE, 16)]
            s_v[pl.ds(c * CHUNK + gbase, 16)] = -acc
            return 0

        lax.fori_loop(0, CHUNK // 16, group_body, 0)

    pending = fire(0, 0)
    for c in range(N_CHUNKS):
        nxt = fire(c + 1, (c + 1) % 2) if c + 1 < N_CHUNKS else None
        for cp in pending:
            cp.wait()
        compute(c, c % 2)
        pending = nxt

    pltpu.sync_copy(s_v, out_hbm.at[pl.ds(base, ROWS_PER_W)])


_sc_scores = functools.partial(
    pl.kernel,
    mesh=plsc.VectorSubcoreMesh(core_axis_name="c", subcore_axis_name="s"),
    out_type=jax.ShapeDtypeStruct((B,), jnp.float32),
    compiler_params=pltpu.CompilerParams(
        needs_layout_passes=False, use_tc_tiling_on_sc=False),
    scratch_types=[
        pltpu.VMEM((ROWS_PER_W,), jnp.int32),
        pltpu.VMEM((ROWS_PER_W,), jnp.int32),
        pltpu.VMEM((ROWS_PER_W,), jnp.int32),
        pltpu.VMEM((CHUNK, DP), jnp.float32),
        pltpu.VMEM((CHUNK, DP), jnp.float32),
        pltpu.VMEM((CHUNK, DP), jnp.float32),
        pltpu.VMEM((CHUNK, DP), jnp.float32),
        pltpu.VMEM((CHUNK, DP), jnp.float32),
        pltpu.VMEM((CHUNK, DP), jnp.float32),
        pltpu.VMEM((16 * TSTRIDE,), jnp.float32),
        pltpu.VMEM((ROWS_PER_W,), jnp.float32),
        pltpu.SemaphoreType.DMA,
        pltpu.SemaphoreType.DMA,
    ],
)(_sc_scores_body)


def _loss_body(x_ref, o_ref):
    x = x_ref[...]
    p = x[:128, :]
    n = x[128:, :]

    def softplus(v):
        return jnp.maximum(v, 0.0) + jnp.log1p(jnp.exp(-jnp.abs(v)))

    tot = jnp.sum(softplus(-p)) + jnp.sum(softplus(n))
    o_ref[0] = tot / (2.0 * (B // 2))


_loss = pl.pallas_call(
    _loss_body,
    out_shape=jax.ShapeDtypeStruct((1,), jnp.float32),
    in_specs=[pl.BlockSpec(memory_space=pltpu.VMEM)],
    out_specs=pl.BlockSpec(memory_space=pltpu.SMEM),
)


def kernel(data, ent_emb, rel_emb):
    ent128, rel128 = _sc_pack(ent_emb.T, rel_emb.T)
    ent_packed = lax.dynamic_update_slice(ent128, ent_emb[V_CUT:], (V_CUT, 0))
    rel_packed = lax.dynamic_update_slice(rel128, rel_emb[V_CUT:], (V_CUT, 0))
    score = _sc_scores(data[0], data[1], data[2], ent_packed, rel_packed)
    loss = _loss(score.reshape(B // 128, 128))[0]
    return loss.reshape(())
